# baseline (speedup 1.0000x reference)
import functools

import jax
import jax.numpy as jnp
from jax import lax
from jax.experimental import pallas as pl
from jax.experimental.pallas import tpu as pltpu
from jax.experimental.pallas import tpu_sc as plsc

VOCAB = 100000
D = 128
B = 64
T = 3072
N = B * T
NC = 2
NS = 16
NW = NC * NS
PER_W = N // NW
CHUNK = 128
NCHUNK = PER_W // CHUNK
POS_CHUNKS = T // CHUNK
POS_PER_SUB = T // NS


NBUF = 4


def _body(idx_hbm, wte_hbm, row_hbm, col_hbm, chan_hbm, out_hbm,
          row_v, col_v, chan_v, pos_build, pos_sh, idx_v, bufs,
          psems, gsems, wsems):
    c = lax.axis_index("c")
    s = lax.axis_index("s")
    w = s * NC + c
    base = w * PER_W

    rcp = pltpu.async_copy(row_hbm, row_v, psems[0])
    ccp = pltpu.async_copy(col_hbm, col_v, psems[1])
    hcp = pltpu.async_copy(chan_hbm, chan_v, psems[2])
    icp = pltpu.async_copy(idx_hbm.at[pl.ds(w * NCHUNK, NCHUNK)], idx_v,
                           psems[3])
    rcp.wait()
    ccp.wait()
    hcp.wait()
    t0 = s * POS_PER_SUB
    for j in range(POS_PER_SUB // 32):
        tb = t0 + j * 32
        chan_i = tb // 1024
        row_i = (tb % 1024) // 32
        rcs = [row_v[row_i, pl.ds(d * 16, 16)] + chan_v[chan_i, pl.ds(d * 16, 16)]
               for d in range(8)]

        def blk(i, carry):
            for d in range(8):
                pos_build[j * 32 + i, pl.ds(d * 16, 16)] = (
                    col_v[i, pl.ds(d * 16, 16)] + rcs[d])
            return carry

        lax.fori_loop(0, 32, blk, 0)
    pltpu.sync_copy(pos_build, pos_sh.at[pl.ds(t0, POS_PER_SUB)])
    icp.wait()
    plsc.subcore_barrier()


    rot = s * (NCHUNK // NS)

    def start_pos(c):
        cc = (c + rot) % NCHUNK
        return pltpu.async_copy(
            pos_sh.at[pl.ds((cc % POS_CHUNKS) * CHUNK, CHUNK)],
            bufs[c % NBUF], psems[c % NBUF])

    def start_gather(c):
        cc = (c + rot) % NCHUNK
        return pltpu.async_copy(wte_hbm.at[idx_v.at[cc]], bufs[c % NBUF],
                                gsems[c % NBUF], add=True)

    def start_write(c):
        cc = (c + rot) % NCHUNK
        return pltpu.async_copy(
            bufs[c % NBUF], out_hbm.at[pl.ds(base + cc * CHUNK, CHUNK)],
            wsems[c % NBUF])

    pos_cp = [None] * NCHUNK
    g_cp = [None] * NCHUNK
    w_cp = [None] * NCHUNK
    pos_cp[0] = start_pos(0)
    pos_cp[1] = start_pos(1)
    for c in range(NCHUNK):
        pos_cp[c].wait()
        g_cp[c] = start_gather(c)
        if c - 1 >= 0:
            g_cp[c - 1].wait()
            w_cp[c - 1] = start_write(c - 1)
        if c + 2 < NCHUNK:
            if c - 2 >= 0:
                w_cp[c - 2].wait()
            pos_cp[c + 2] = start_pos(c + 2)
    g_cp[NCHUNK - 1].wait()
    w_cp[NCHUNK - 1] = start_write(NCHUNK - 1)
    for c in range(NCHUNK - 4, NCHUNK):
        w_cp[c].wait()


@jax.jit
def _run(idx2, wte, row_w, col_w, chan_w):
    mesh = plsc.VectorSubcoreMesh(core_axis_name="c", subcore_axis_name="s",
                                  num_cores=NC, num_subcores=NS)
    f = pl.kernel(
        _body,
        out_type=jax.ShapeDtypeStruct((N, D), jnp.float32),
        mesh=mesh,
        scratch_types=[
            pltpu.VMEM((32, D), jnp.float32),
            pltpu.VMEM((32, D), jnp.float32),
            pltpu.VMEM((3, D), jnp.float32),
            pltpu.VMEM((POS_PER_SUB, D), jnp.float32),
            pltpu.VMEM_SHARED((T, D), jnp.float32),
            pltpu.VMEM((NCHUNK, CHUNK), jnp.int32),
            [pltpu.VMEM((CHUNK, D), jnp.float32) for _ in range(NBUF)],
            [pltpu.SemaphoreType.DMA for _ in range(NBUF)],
            [pltpu.SemaphoreType.DMA for _ in range(NBUF)],
            [pltpu.SemaphoreType.DMA for _ in range(NBUF)],
        ],
    )
    return f(idx2, wte, row_w, col_w, chan_w)


def kernel(idx, wte, row_w, col_w, chan_w):
    idx2 = idx.reshape(N // CHUNK, CHUNK).astype(jnp.int32)
    out = _run(idx2, wte, row_w, col_w, chan_w)
    return out.reshape(B, T, D)


# speedup vs baseline: 1.0189x; 1.0189x over previous
import jax
import jax.numpy as jnp
from jax import lax
from jax.experimental import pallas as pl
from jax.experimental.pallas import tpu as pltpu
from jax.experimental.pallas import tpu_sc as plsc

VOCAB = 100000
D = 128
B = 64
T = 3072
N = B * T
NC = 2
NS = 16
NW = NC * NS
PER_W = N // NW
CHUNK = 128
NCHUNK = PER_W // CHUNK
POS_CHUNKS = T // CHUNK
POS_PER_SUB = T // NS


NBUF = 4


def _body(idx_hbm, wte_hbm, row_hbm, col_hbm, chan_hbm, out_hbm,
          row_v, col_v, chan_v, pos_build, pos_sh, idx_v, bufs,
          psems, gsems, wsems):
    c = lax.axis_index("c")
    s = lax.axis_index("s")
    w = s * NC + c
    base = w * PER_W

    rcp = pltpu.async_copy(row_hbm, row_v, psems[0])
    ccp = pltpu.async_copy(col_hbm, col_v, psems[1])
    hcp = pltpu.async_copy(chan_hbm, chan_v, psems[2])
    icp = pltpu.async_copy(idx_hbm.at[pl.ds(w * NCHUNK, NCHUNK)], idx_v,
                           psems[3])
    rcp.wait()
    ccp.wait()
    hcp.wait()
    t0 = s * POS_PER_SUB
    for j in range(POS_PER_SUB // 32):
        tb = t0 + j * 32
        chan_i = tb // 1024
        row_i = (tb % 1024) // 32
        rcs = [row_v[row_i, pl.ds(d * 16, 16)] + chan_v[chan_i, pl.ds(d * 16, 16)]
               for d in range(8)]

        def blk(i, carry):
            for d in range(8):
                pos_build[j * 32 + i, pl.ds(d * 16, 16)] = (
                    col_v[i, pl.ds(d * 16, 16)] + rcs[d])
            return carry

        lax.fori_loop(0, 32, blk, 0)
    pltpu.sync_copy(pos_build, pos_sh.at[pl.ds(t0, POS_PER_SUB)])
    icp.wait()
    plsc.subcore_barrier()


    def start_pos(c):
        return pltpu.async_copy(
            pos_sh.at[pl.ds((c % POS_CHUNKS) * CHUNK, CHUNK)],
            bufs[c % NBUF], psems[c % NBUF])

    def start_gather(c):
        return pltpu.async_copy(wte_hbm.at[idx_v.at[c]], bufs[c % NBUF],
                                gsems[c % NBUF], add=True)

    def start_write(c):
        return pltpu.async_copy(
            bufs[c % NBUF], out_hbm.at[pl.ds(base + c * CHUNK, CHUNK)],
            wsems[c % NBUF])

    pos_cp = [None] * NCHUNK
    g_cp = [None] * NCHUNK
    w_cp = [None] * NCHUNK
    pos_cp[0] = start_pos(0)
    pos_cp[1] = start_pos(1)
    for c in range(NCHUNK):
        pos_cp[c].wait()
        g_cp[c] = start_gather(c)
        if c - 1 >= 0:
            g_cp[c - 1].wait()
            w_cp[c - 1] = start_write(c - 1)
        if c + 2 < NCHUNK:
            if c - 2 >= 0:
                w_cp[c - 2].wait()
            pos_cp[c + 2] = start_pos(c + 2)
    g_cp[NCHUNK - 1].wait()
    w_cp[NCHUNK - 1] = start_write(NCHUNK - 1)
    for c in range(NCHUNK - 4, NCHUNK):
        w_cp[c].wait()


@jax.jit
def _run(idx2, wte, row_w, col_w, chan_w):
    mesh = plsc.VectorSubcoreMesh(core_axis_name="c", subcore_axis_name="s",
                                  num_cores=NC, num_subcores=NS)
    f = pl.kernel(
        _body,
        out_type=jax.ShapeDtypeStruct((N, D), jnp.float32),
        mesh=mesh,
        scratch_types=[
            pltpu.VMEM((32, D), jnp.float32),
            pltpu.VMEM((32, D), jnp.float32),
            pltpu.VMEM((3, D), jnp.float32),
            pltpu.VMEM((POS_PER_SUB, D), jnp.float32),
            pltpu.VMEM_SHARED((T, D), jnp.float32),
            pltpu.VMEM((NCHUNK, CHUNK), jnp.int32),
            [pltpu.VMEM((CHUNK, D), jnp.float32) for _ in range(NBUF)],
            [pltpu.SemaphoreType.DMA for _ in range(NBUF)],
            [pltpu.SemaphoreType.DMA for _ in range(NBUF)],
            [pltpu.SemaphoreType.DMA for _ in range(NBUF)],
        ],
    )
    return f(idx2, wte, row_w, col_w, chan_w)


def kernel(idx, wte, row_w, col_w, chan_w):
    idx2 = idx.reshape(N // CHUNK, CHUNK).astype(jnp.int32)
    out = _run(idx2, wte, row_w, col_w, chan_w)
    return out.reshape(B, T, D)
